# baseline (speedup 1.0000x reference)
import functools

import jax
import jax.numpy as jnp
from jax import lax
from jax.experimental import pallas as pl
from jax.experimental.pallas import tpu as pltpu
from jax.experimental.pallas import tpu_sc as plsc

N = 10000
NP = 10112
E = 640000
IN = 128
H = 4
C = 32
HC = H * C
HP = 8
WD = HC + 16
NC, NS, L = 2, 16, 16
NW = NC * NS
K = 128
NCH0 = 160
NCH1 = 160
TOTCH = NS * (NCH0 + NCH1)
TOTCHA = TOTCH + 1
EP = TOTCH * K
EPA = TOTCHA * K
ROWS_PER_TILE = NP // NS
PAD_DST = N + 100

_f32 = jnp.float32


def _ewsum_body(ew_ref, out_ref):
    @pl.when(pl.program_id(0) == 0)
    def _():
        out_ref[...] = jnp.zeros_like(out_ref)

    out_ref[...] = out_ref[...] + jnp.sum(ew_ref[...]).reshape(1, 1)


def _mean_ew(ew):
    ew2 = ew.reshape(5000, 128)
    s = pl.pallas_call(
        _ewsum_body,
        grid=(5,),
        in_specs=[pl.BlockSpec((1000, 128), lambda i: (i, 0))],
        out_specs=pl.BlockSpec((1, 1), lambda i: (0, 0)),
        out_shape=jax.ShapeDtypeStruct((1, 1), _f32),
    )(ew2)
    return s / float(E)


def _prep_body(x_ref, w_ref, msrc_ref, mdst_ref, wprod_ref, sel_ref,
               meanw_ref, hs_ref, adst_ref, wc_ref, sn_ref):
    h = jnp.dot(x_ref[...], w_ref[...], preferred_element_type=_f32)
    asrc = jnp.dot(h, msrc_ref[...], preferred_element_type=_f32)
    adst = jnp.dot(h, mdst_ref[...], preferred_element_type=_f32)
    hs_ref[...] = jnp.concatenate([h, asrc, asrc], axis=1)
    adst_ref[...] = jnp.concatenate([adst, adst], axis=1)
    wc = jnp.dot(wprod_ref[...], sel_ref[...], preferred_element_type=_f32)
    wc_ref[...] = wc
    al = asrc + adst + meanw_ref[0, 0] * wc
    al = jnp.maximum(al, 0.2 * al)
    tl = jnp.exp(al)
    tlb = jnp.dot(tl, jnp.transpose(sel_ref[...]),
                  preferred_element_type=_f32)
    sn_ref[...] = 0.5 * jnp.concatenate([h * tlb, tl, tl], axis=1)


def _prep(xp, w, msrc, mdst, wprod, sel, meanw):
    nblk = 8
    blk = NP // nblk
    return pl.pallas_call(
        _prep_body,
        grid=(nblk,),
        in_specs=[
            pl.BlockSpec((blk, IN), lambda i: (i, 0)),
            pl.BlockSpec((IN, HC), lambda i: (0, 0)),
            pl.BlockSpec((HC, HP), lambda i: (0, 0)),
            pl.BlockSpec((HC, HP), lambda i: (0, 0)),
            pl.BlockSpec((1, HC), lambda i: (0, 0)),
            pl.BlockSpec((HC, HP), lambda i: (0, 0)),
            pl.BlockSpec((1, 1), lambda i: (0, 0)),
        ],
        out_specs=[
            pl.BlockSpec((blk, WD), lambda i: (i, 0)),
            pl.BlockSpec((blk, L), lambda i: (i, 0)),
            pl.BlockSpec((1, HP), lambda i: (0, 0)),
            pl.BlockSpec((blk, WD), lambda i: (i, 0)),
        ],
        out_shape=[
            jax.ShapeDtypeStruct((NP, WD), _f32),
            jax.ShapeDtypeStruct((NP, L), _f32),
            jax.ShapeDtypeStruct((1, HP), _f32),
            jax.ShapeDtypeStruct((NP, WD), _f32),
        ],
    )(xp, w, msrc, mdst, wprod, sel, meanw)


def _sc_edge_body(sd_hbm, ew_hbm, hs_hbm, adst_hbm,
                  wc_hbm, sn_hbm, acc_out,
                  sd0, sd1, ewb0, ewb1,
                  hsg, adg, scat,
                  wcv, acc,
                  semi0, semi1, semg0, sems0):
    sdb = (sd0, sd1)
    ewb = (ewb0, ewb1)
    semi = (semi0, semi1)

    cid = lax.axis_index("c")
    sid = lax.axis_index("s")
    rbase = sid * ROWS_PER_TILE

    pltpu.sync_copy(sn_hbm.at[pl.ds(rbase, ROWS_PER_TILE)],
                    acc.at[pl.ds(rbase, ROWS_PER_TILE)])
    pltpu.sync_copy(wc_hbm, wcv)
    plsc.subcore_barrier()

    wc16 = wcv[...]
    nch = jnp.where(cid == 0, NCH0, NCH1)
    cbase = cid * (NS * NCH0) + sid * nch

    def issue_idx(j, s2):
        g = cbase + j
        pltpu.async_copy(sd_hbm.at[g], sdb[s2], semi[s2])
        pltpu.async_copy(ew_hbm.at[g], ewb[s2], semi[s2])

    def wait_idx(j, s2):
        g = cbase + j
        pltpu.make_async_copy(sd_hbm.at[g], sdb[s2], semi[s2]).wait()
        pltpu.make_async_copy(ew_hbm.at[g], ewb[s2], semi[s2]).wait()

    def issue_gathers(s2):
        pltpu.async_copy(hs_hbm.at[sdb[s2].at[0]], hsg, semg0)
        pltpu.async_copy(adst_hbm.at[sdb[s2].at[1]], adg, semg0)

    def wait_gathers(s2):
        pltpu.make_async_copy(hs_hbm.at[sdb[s2].at[0]], hsg,
                              semg0).wait()
        pltpu.make_async_copy(adst_hbm.at[sdb[s2].at[1]], adg,
                              semg0).wait()

    def issue_scatter(s2):
        pltpu.async_copy(scat, acc.at[sdb[s2].at[1]], sems0, add=True)

    def wait_scatter(s2):
        pltpu.make_async_copy(scat, acc.at[sdb[s2].at[1]], sems0).wait()

    def compute(s2):
        def t_body(g, _):
            ewv = ewb[s2][pl.ds(g * L, L)]
            for j in range(L):
                e = g * L + j
                a = hsg[e, pl.ds(HC, L)] + adg[e, :] + ewv[j] * wc16
                a = jnp.maximum(a, 0.2 * a)
                t = jnp.exp(a)
                scat[e, pl.ds(HC, L)] = t
                for half in range(4):
                    ts = t[half]
                    scat[e, pl.ds(half * 32, 16)] = \
                        hsg[e, pl.ds(half * 32, 16)] * ts
                    scat[e, pl.ds(half * 32 + 16, 16)] = \
                        hsg[e, pl.ds(half * 32 + 16, 16)] * ts
            return 0

        lax.fori_loop(0, K // L, t_body, 0)

    issue_idx(0, 0)
    wait_idx(0, 0)
    issue_gathers(0)
    issue_idx(1, 1)
    wait_gathers(0)
    compute(0)
    issue_scatter(0)

    def pair(b, _):
        for u in range(2):
            i = 1 + b * 2 + u
            s2 = (1 + u) % 2
            wait_idx(i, s2)
            issue_gathers(s2)
            wait_scatter(1 - s2)
            issue_idx(i + 1, 1 - s2)
            wait_gathers(s2)
            compute(s2)
            issue_scatter(s2)
        return 0

    lax.fori_loop(0, (nch - 2) // 2, pair, 0)

    wait_idx(nch - 1, 1)
    issue_gathers(1)
    wait_scatter(0)
    wait_gathers(1)
    compute(1)
    issue_scatter(1)
    wait_scatter(1)

    plsc.subcore_barrier()
    pltpu.sync_copy(acc.at[pl.ds(rbase, ROWS_PER_TILE)],
                    acc_out.at[cid, pl.ds(rbase, ROWS_PER_TILE)])


_sc_edge = functools.partial(
    pl.kernel,
    _sc_edge_body,
    out_type=jax.ShapeDtypeStruct((NC, NP, WD), _f32),
    mesh=plsc.VectorSubcoreMesh(core_axis_name="c", subcore_axis_name="s",
                                num_cores=NC, num_subcores=NS),
    compiler_params=pltpu.CompilerParams(use_tc_tiling_on_sc=False),
    scratch_types=(
        [pltpu.VMEM((2, K), jnp.int32)] * 2
        + [pltpu.VMEM((K,), _f32)] * 2
        + [pltpu.VMEM((K, WD), _f32)]
        + [pltpu.VMEM((K, L), _f32)]
        + [pltpu.VMEM((K, WD), _f32)]
        + [pltpu.VMEM((L,), _f32)]
        + [pltpu.VMEM_SHARED((NP, WD), _f32)]
        + [pltpu.SemaphoreType.DMA] * 4
    ),
)()


def _combine_body(acc_ref, sel_ref, b_ref, out_ref):
    asum = acc_ref[0] + acc_ref[1]
    dsum = asum[:, HC:HC + HP]
    denb = jnp.dot(dsum, jnp.transpose(sel_ref[...]),
                   preferred_element_type=_f32)
    out_ref[...] = asum[:, :HC] / denb + b_ref[...]


def _combine(accs, sel, b2d):
    nblk = 8
    blk = NP // nblk
    return pl.pallas_call(
        _combine_body,
        grid=(nblk,),
        in_specs=[
            pl.BlockSpec((NC, blk, WD), lambda i: (0, i, 0)),
            pl.BlockSpec((HC, HP), lambda i: (0, 0)),
            pl.BlockSpec((1, HC), lambda i: (0, 0)),
        ],
        out_specs=pl.BlockSpec((blk, HC), lambda i: (i, 0)),
        out_shape=jax.ShapeDtypeStruct((NP, HC), _f32),
    )(accs, sel, b2d)


def _layer(xp, sd3, ew2, w, att_src, att_dst, w_edge, att_edge,
           bias, sel, meanw):
    msrc = sel * att_src.reshape(-1)[:, None]
    mdst = sel * att_dst.reshape(-1)[:, None]
    wprod = (w_edge.reshape(1, HC) * att_edge.reshape(1, HC))
    hs, adst, wc, sn = _prep(xp, w, msrc, mdst, wprod, sel, meanw)
    accs = _sc_edge(sd3, ew2, hs, adst, jnp.tile(wc.reshape(HP), 2), sn)
    return _combine(accs, sel, bias.reshape(1, HC))


def kernel(x, edge_index, edge_weight, W1, att_src1, att_dst1, W_edge1,
           att_edge1, bias1, W2, att_src2, att_dst2, W_edge2, att_edge2,
           bias2):
    src, dst = edge_index[0], edge_index[1]
    pad = EPA - E
    src_p = jnp.concatenate([src, jnp.zeros((pad,), jnp.int32)])
    dst_p = jnp.concatenate([dst, jnp.full((pad,), PAD_DST, jnp.int32)])
    ew_p = jnp.concatenate([edge_weight, jnp.zeros((pad,), _f32)])
    sd3 = jnp.stack([src_p.reshape(TOTCHA, K), dst_p.reshape(TOTCHA, K)],
                    axis=1)
    ew2 = ew_p.reshape(TOTCHA, K)
    xp = jnp.pad(x, ((0, NP - N), (0, 0)))
    sel = (jnp.arange(HC)[:, None] // C == jnp.arange(HP)[None, :]
           ).astype(_f32)
    meanw = _mean_ew(edge_weight)
    out1 = _layer(xp, sd3, ew2, W1, att_src1, att_dst1, W_edge1,
                  att_edge1, bias1, sel, meanw)
    out2 = _layer(out1, sd3, ew2, W2, att_src2, att_dst2, W_edge2,
                  att_edge2, bias2, sel, meanw)
    return out2[:N]


# speedup vs baseline: 1.2649x; 1.2649x over previous
import functools

import jax
import jax.numpy as jnp
from jax import lax
from jax.experimental import pallas as pl
from jax.experimental.pallas import tpu as pltpu
from jax.experimental.pallas import tpu_sc as plsc

N = 10000
NP = 10240
E = 640000
IN = 128
H = 4
C = 32
HC = H * C
HP = 8
NC, NS, L = 2, 16, 16
NW = NC * NS
K = 128
NCH = 158
EW = NCH * K
EP = EW * NW
ROWS_PER_TILE = NP // NS
PAD_DST = N + 100

_f32 = jnp.float32


def _ewsum_body(ew_ref, out_ref):
    @pl.when(pl.program_id(0) == 0)
    def _():
        out_ref[...] = jnp.zeros_like(out_ref)

    out_ref[...] = out_ref[...] + jnp.sum(ew_ref[...]).reshape(1, 1)


def _mean_ew(ew):
    ew2 = ew.reshape(5000, 128)
    s = pl.pallas_call(
        _ewsum_body,
        grid=(5,),
        in_specs=[pl.BlockSpec((1000, 128), lambda i: (i, 0))],
        out_specs=pl.BlockSpec((1, 1), lambda i: (0, 0)),
        out_shape=jax.ShapeDtypeStruct((1, 1), _f32),
    )(ew2)
    return s / float(E)


def _prep_body(x_ref, w_ref, msrc_ref, mdst_ref, wprod_ref, sel_ref,
               meanw_ref, h_ref, asrc_ref, adst_ref, wc_ref, snh_ref,
               sdh_ref):
    h = jnp.dot(x_ref[...], w_ref[...], preferred_element_type=_f32)
    h_ref[...] = h
    asrc = jnp.dot(h, msrc_ref[...], preferred_element_type=_f32)
    adst = jnp.dot(h, mdst_ref[...], preferred_element_type=_f32)
    asrc_ref[...] = jnp.concatenate([asrc, asrc], axis=1)
    adst_ref[...] = jnp.concatenate([adst, adst], axis=1)
    wc = jnp.dot(wprod_ref[...], sel_ref[...], preferred_element_type=_f32)
    wc_ref[...] = wc
    al = asrc + adst + meanw_ref[0, 0] * wc
    al = jnp.maximum(al, 0.2 * al)
    tl = jnp.exp(al)
    tlb = jnp.dot(tl, jnp.transpose(sel_ref[...]),
                  preferred_element_type=_f32)
    snh_ref[...] = 0.5 * h * tlb
    sdh_ref[...] = 0.5 * jnp.concatenate([tl, tl], axis=1)


def _prep(xp, w, msrc, mdst, wprod, sel, meanw):
    nblk = 8
    blk = NP // nblk
    return pl.pallas_call(
        _prep_body,
        grid=(nblk,),
        in_specs=[
            pl.BlockSpec((blk, IN), lambda i: (i, 0)),
            pl.BlockSpec((IN, HC), lambda i: (0, 0)),
            pl.BlockSpec((HC, HP), lambda i: (0, 0)),
            pl.BlockSpec((HC, HP), lambda i: (0, 0)),
            pl.BlockSpec((1, HC), lambda i: (0, 0)),
            pl.BlockSpec((HC, HP), lambda i: (0, 0)),
            pl.BlockSpec((1, 1), lambda i: (0, 0)),
        ],
        out_specs=[
            pl.BlockSpec((blk, HC), lambda i: (i, 0)),
            pl.BlockSpec((blk, L), lambda i: (i, 0)),
            pl.BlockSpec((blk, L), lambda i: (i, 0)),
            pl.BlockSpec((1, HP), lambda i: (0, 0)),
            pl.BlockSpec((blk, HC), lambda i: (i, 0)),
            pl.BlockSpec((blk, L), lambda i: (i, 0)),
        ],
        out_shape=[
            jax.ShapeDtypeStruct((NP, HC), _f32),
            jax.ShapeDtypeStruct((NP, L), _f32),
            jax.ShapeDtypeStruct((NP, L), _f32),
            jax.ShapeDtypeStruct((1, HP), _f32),
            jax.ShapeDtypeStruct((NP, HC), _f32),
            jax.ShapeDtypeStruct((NP, L), _f32),
        ],
    )(xp, w, msrc, mdst, wprod, sel, meanw)


def _sc_edge_body(src_hbm, dst_hbm, ew_hbm, asrc_hbm, adst_hbm, h_hbm,
                  wc_hbm, snh_hbm, sdh_hbm, num_out, den_out,
                  sidx0, sidx1, didx0, didx1, ewb0, ewb1,
                  asg, adg, hg, tb, wcv, accnum, accden,
                  semi0, semi1, semg0, sems0):
    sidx = (sidx0, sidx1)
    didx = (didx0, didx1)
    ewb = (ewb0, ewb1)
    semi = (semi0, semi1)

    cid = lax.axis_index("c")
    sid = lax.axis_index("s")
    wid = sid * NC + cid
    rbase = sid * ROWS_PER_TILE

    pltpu.sync_copy(snh_hbm.at[pl.ds(rbase, ROWS_PER_TILE)],
                    accnum.at[pl.ds(rbase, ROWS_PER_TILE)])
    pltpu.sync_copy(sdh_hbm.at[pl.ds(rbase, ROWS_PER_TILE)],
                    accden.at[pl.ds(rbase, ROWS_PER_TILE)])
    pltpu.sync_copy(wc_hbm, wcv)
    plsc.subcore_barrier()

    wc16 = wcv[...]
    ebase = wid * EW

    def issue_idx(j, s2):
        off = ebase + j * K
        pltpu.async_copy(src_hbm.at[pl.ds(off, K)], sidx[s2], semi[s2])
        pltpu.async_copy(dst_hbm.at[pl.ds(off, K)], didx[s2], semi[s2])
        pltpu.async_copy(ew_hbm.at[pl.ds(off, K)], ewb[s2], semi[s2])

    def wait_idx(j, s2):
        off = ebase + j * K
        pltpu.make_async_copy(src_hbm.at[pl.ds(off, K)], sidx[s2],
                              semi[s2]).wait()
        pltpu.make_async_copy(dst_hbm.at[pl.ds(off, K)], didx[s2],
                              semi[s2]).wait()
        pltpu.make_async_copy(ew_hbm.at[pl.ds(off, K)], ewb[s2],
                              semi[s2]).wait()

    def issue_gathers(s2):
        pltpu.async_copy(asrc_hbm.at[sidx[s2]], asg, semg0)
        pltpu.async_copy(adst_hbm.at[didx[s2]], adg, semg0)
        pltpu.async_copy(h_hbm.at[sidx[s2]], hg, semg0)

    def wait_gathers(s2):
        pltpu.make_async_copy(asrc_hbm.at[sidx[s2]], asg, semg0).wait()
        pltpu.make_async_copy(adst_hbm.at[didx[s2]], adg, semg0).wait()
        pltpu.make_async_copy(h_hbm.at[sidx[s2]], hg, semg0).wait()

    def issue_scatter(s2):
        pltpu.async_copy(hg, accnum.at[didx[s2]], sems0, add=True)
        pltpu.async_copy(tb, accden.at[didx[s2]], sems0, add=True)

    def wait_scatter(s2):
        pltpu.make_async_copy(hg, accnum.at[didx[s2]], sems0).wait()
        pltpu.make_async_copy(tb, accden.at[didx[s2]], sems0).wait()

    def compute(s2):
        def t_body(g, _):
            ewv = ewb[s2][pl.ds(g * L, L)]
            for j in range(L):
                e = g * L + j
                a = asg[e, :] + adg[e, :] + ewv[j] * wc16
                a = jnp.maximum(a, 0.2 * a)
                t = jnp.exp(a)
                tb[e, :] = t
                for half in range(4):
                    ts = t[half]
                    hg[e, pl.ds(half * 32, 16)] = \
                        hg[e, pl.ds(half * 32, 16)] * ts
                    hg[e, pl.ds(half * 32 + 16, 16)] = \
                        hg[e, pl.ds(half * 32 + 16, 16)] * ts
            return 0

        lax.fori_loop(0, K // L, t_body, 0)

    issue_idx(0, 0)
    wait_idx(0, 0)
    issue_gathers(0)
    issue_idx(1, 1)
    wait_gathers(0)
    compute(0)
    issue_scatter(0)

    def pair(b, _):
        for u in range(2):
            i = 1 + b * 2 + u
            s2 = (1 + u) % 2
            wait_idx(i, s2)
            issue_gathers(s2)
            wait_scatter(1 - s2)
            issue_idx(i + 1, 1 - s2)
            wait_gathers(s2)
            compute(s2)
            issue_scatter(s2)
        return 0

    lax.fori_loop(0, (NCH - 2) // 2, pair, 0)

    wait_idx(NCH - 1, 1)
    issue_gathers(1)
    wait_scatter(0)
    wait_gathers(1)
    compute(1)
    issue_scatter(1)
    wait_scatter(1)

    plsc.subcore_barrier()
    pltpu.sync_copy(accnum.at[pl.ds(rbase, ROWS_PER_TILE)],
                    num_out.at[cid, pl.ds(rbase, ROWS_PER_TILE)])
    pltpu.sync_copy(accden.at[pl.ds(rbase, ROWS_PER_TILE)],
                    den_out.at[cid, pl.ds(rbase, ROWS_PER_TILE)])


_sc_edge = functools.partial(
    pl.kernel,
    _sc_edge_body,
    out_type=(jax.ShapeDtypeStruct((NC, NP, HC), _f32),
              jax.ShapeDtypeStruct((NC, NP, L), _f32)),
    mesh=plsc.VectorSubcoreMesh(core_axis_name="c", subcore_axis_name="s",
                                num_cores=NC, num_subcores=NS),
    compiler_params=pltpu.CompilerParams(use_tc_tiling_on_sc=False),
    scratch_types=(
        [pltpu.VMEM((K,), jnp.int32)] * 2
        + [pltpu.VMEM((K,), jnp.int32)] * 2
        + [pltpu.VMEM((K,), _f32)] * 2
        + [pltpu.VMEM((K, L), _f32)]
        + [pltpu.VMEM((K, L), _f32)]
        + [pltpu.VMEM((K, HC), _f32)]
        + [pltpu.VMEM((K, L), _f32)]
        + [pltpu.VMEM((L,), _f32)]
        + [pltpu.VMEM_SHARED((NP, HC), _f32)]
        + [pltpu.VMEM_SHARED((NP, L), _f32)]
        + [pltpu.SemaphoreType.DMA] * 4
    ),
)()


def _combine_body(num_ref, den_ref, sel_ref, b_ref, out_ref):
    dsum = (den_ref[0] + den_ref[1])[:, :HP]
    denb = jnp.dot(dsum, jnp.transpose(sel_ref[...]),
                   preferred_element_type=_f32)
    out_ref[...] = (num_ref[0] + num_ref[1]) / denb + b_ref[...]


def _combine(num, den, sel, b2d):
    nblk = 8
    blk = NP // nblk
    return pl.pallas_call(
        _combine_body,
        grid=(nblk,),
        in_specs=[
            pl.BlockSpec((NC, blk, HC), lambda i: (0, i, 0)),
            pl.BlockSpec((NC, blk, L), lambda i: (0, i, 0)),
            pl.BlockSpec((HC, HP), lambda i: (0, 0)),
            pl.BlockSpec((1, HC), lambda i: (0, 0)),
        ],
        out_specs=pl.BlockSpec((blk, HC), lambda i: (i, 0)),
        out_shape=jax.ShapeDtypeStruct((NP, HC), _f32),
    )(num, den, sel, b2d)


def _layer(xp, src_p, dst_p, ew_p, w, att_src, att_dst, w_edge, att_edge,
           bias, sel, meanw):
    msrc = sel * att_src.reshape(-1)[:, None]
    mdst = sel * att_dst.reshape(-1)[:, None]
    wprod = (w_edge.reshape(1, HC) * att_edge.reshape(1, HC))
    h, asrc, adst, wc, snh, sdh = _prep(xp, w, msrc, mdst, wprod, sel, meanw)
    num, den = _sc_edge(src_p, dst_p, ew_p, asrc, adst, h,
                        jnp.tile(wc.reshape(HP), 2), snh, sdh)
    return _combine(num, den, sel, bias.reshape(1, HC))


def kernel(x, edge_index, edge_weight, W1, att_src1, att_dst1, W_edge1,
           att_edge1, bias1, W2, att_src2, att_dst2, W_edge2, att_edge2,
           bias2):
    src, dst = edge_index[0], edge_index[1]
    pad = EP - E
    src_p = jnp.concatenate([src, jnp.zeros((pad,), jnp.int32)])
    dst_p = jnp.concatenate([dst, jnp.full((pad,), PAD_DST, jnp.int32)])
    ew_p = jnp.concatenate([edge_weight, jnp.zeros((pad,), _f32)])
    xp = jnp.pad(x, ((0, NP - N), (0, 0)))
    sel = (jnp.arange(HC)[:, None] // C == jnp.arange(HP)[None, :]
           ).astype(_f32)
    meanw = _mean_ew(edge_weight)
    out1 = _layer(xp, src_p, dst_p, ew_p, W1, att_src1, att_dst1, W_edge1,
                  att_edge1, bias1, sel, meanw)
    out2 = _layer(out1, src_p, dst_p, ew_p, W2, att_src2, att_dst2, W_edge2,
                  att_edge2, bias2, sel, meanw)
    return out2[:N]


# speedup vs baseline: 1.3009x; 1.0284x over previous
import functools

import jax
import jax.numpy as jnp
from jax import lax
from jax.experimental import pallas as pl
from jax.experimental.pallas import tpu as pltpu
from jax.experimental.pallas import tpu_sc as plsc

N = 10000
NP = 10240
E = 640000
IN = 128
H = 4
C = 32
HC = H * C
HP = 8
NC, NS, L = 2, 16, 16
NW = NC * NS
K = 128
NCH = 157
EW = NCH * K
EP = EW * NW
ROWS_PER_TILE = NP // NS
PAD_DST = N + 100

_f32 = jnp.float32


def _ewsum_body(ew_ref, out_ref):
    @pl.when(pl.program_id(0) == 0)
    def _():
        out_ref[...] = jnp.zeros_like(out_ref)

    out_ref[...] = out_ref[...] + jnp.sum(ew_ref[...]).reshape(1, 1)


def _mean_ew(ew):
    ew2 = ew.reshape(5000, 128)
    s = pl.pallas_call(
        _ewsum_body,
        grid=(5,),
        in_specs=[pl.BlockSpec((1000, 128), lambda i: (i, 0))],
        out_specs=pl.BlockSpec((1, 1), lambda i: (0, 0)),
        out_shape=jax.ShapeDtypeStruct((1, 1), _f32),
    )(ew2)
    return s / float(E)


def _prep_body(x_ref, w_ref, msrc_ref, mdst_ref, wprod_ref, sel_ref,
               meanw_ref, h_ref, asrc_ref, adst_ref, wc_ref, snh_ref,
               sdh_ref):
    h = jnp.dot(x_ref[...], w_ref[...], preferred_element_type=_f32)
    h_ref[...] = h
    asrc = jnp.dot(h, msrc_ref[...], preferred_element_type=_f32)
    adst = jnp.dot(h, mdst_ref[...], preferred_element_type=_f32)
    asrc_ref[...] = jnp.concatenate([asrc, asrc], axis=1)
    adst_ref[...] = jnp.concatenate([adst, adst], axis=1)
    wc = jnp.dot(wprod_ref[...], sel_ref[...], preferred_element_type=_f32)
    wc_ref[...] = wc
    al = asrc + adst + meanw_ref[0, 0] * wc
    al = jnp.maximum(al, 0.2 * al)
    tl = jnp.exp(al)
    tlb = jnp.dot(tl, jnp.transpose(sel_ref[...]),
                  preferred_element_type=_f32)
    snh_ref[...] = 0.5 * h * tlb
    sdh_ref[...] = 0.5 * jnp.concatenate([tl, tl], axis=1)


def _prep(xp, w, msrc, mdst, wprod, sel, meanw):
    nblk = 8
    blk = NP // nblk
    return pl.pallas_call(
        _prep_body,
        grid=(nblk,),
        in_specs=[
            pl.BlockSpec((blk, IN), lambda i: (i, 0)),
            pl.BlockSpec((IN, HC), lambda i: (0, 0)),
            pl.BlockSpec((HC, HP), lambda i: (0, 0)),
            pl.BlockSpec((HC, HP), lambda i: (0, 0)),
            pl.BlockSpec((1, HC), lambda i: (0, 0)),
            pl.BlockSpec((HC, HP), lambda i: (0, 0)),
            pl.BlockSpec((1, 1), lambda i: (0, 0)),
        ],
        out_specs=[
            pl.BlockSpec((blk, HC), lambda i: (i, 0)),
            pl.BlockSpec((blk, L), lambda i: (i, 0)),
            pl.BlockSpec((blk, L), lambda i: (i, 0)),
            pl.BlockSpec((1, HP), lambda i: (0, 0)),
            pl.BlockSpec((blk, HC), lambda i: (i, 0)),
            pl.BlockSpec((blk, L), lambda i: (i, 0)),
        ],
        out_shape=[
            jax.ShapeDtypeStruct((NP, HC), _f32),
            jax.ShapeDtypeStruct((NP, L), _f32),
            jax.ShapeDtypeStruct((NP, L), _f32),
            jax.ShapeDtypeStruct((1, HP), _f32),
            jax.ShapeDtypeStruct((NP, HC), _f32),
            jax.ShapeDtypeStruct((NP, L), _f32),
        ],
    )(xp, w, msrc, mdst, wprod, sel, meanw)


def _sc_edge_body(src_hbm, dst_hbm, ew_hbm, asrc_hbm, adst_hbm, h_hbm,
                  wc_hbm, snh_hbm, sdh_hbm, num_out, den_out,
                  sidx, didx, ewb, asg, adg, hg, tb, wcv, accnum, accden,
                  sem0, sem1, sem2):
    cid = lax.axis_index("c")
    sid = lax.axis_index("s")
    wid = sid * NC + cid
    rbase = sid * ROWS_PER_TILE

    pltpu.sync_copy(snh_hbm.at[pl.ds(rbase, ROWS_PER_TILE)],
                    accnum.at[pl.ds(rbase, ROWS_PER_TILE)])
    pltpu.sync_copy(sdh_hbm.at[pl.ds(rbase, ROWS_PER_TILE)],
                    accden.at[pl.ds(rbase, ROWS_PER_TILE)])
    pltpu.sync_copy(wc_hbm, wcv)
    plsc.subcore_barrier()

    wc16 = wcv[...]
    ebase = wid * EW

    def chunk_body(i, carry):
        off = ebase + i * K
        d0 = pltpu.async_copy(src_hbm.at[pl.ds(off, K)], sidx, sem0)
        d1 = pltpu.async_copy(dst_hbm.at[pl.ds(off, K)], didx, sem1)
        d2 = pltpu.async_copy(ew_hbm.at[pl.ds(off, K)], ewb, sem2)
        d0.wait()
        d1.wait()
        d2.wait()
        g0 = pltpu.async_copy(asrc_hbm.at[sidx], asg, sem0)
        g1 = pltpu.async_copy(adst_hbm.at[didx], adg, sem1)
        g2 = pltpu.async_copy(h_hbm.at[sidx], hg, sem2)
        g0.wait()
        g1.wait()
        g2.wait()

        def t_body(g, _):
            ewv = ewb[pl.ds(g * L, L)]
            for j in range(L):
                e = g * L + j
                a = asg[e, :] + adg[e, :] + ewv[j] * wc16
                a = jnp.maximum(a, 0.2 * a)
                t = jnp.exp(a)
                tb[e, :] = t
                for half in range(4):
                    ts = t[half]
                    hg[e, pl.ds(half * 32, 16)] = \
                        hg[e, pl.ds(half * 32, 16)] * ts
                    hg[e, pl.ds(half * 32 + 16, 16)] = \
                        hg[e, pl.ds(half * 32 + 16, 16)] * ts
            return 0

        lax.fori_loop(0, K // L, t_body, 0)

        pltpu.sync_copy(hg, accnum.at[didx], add=True)
        pltpu.sync_copy(tb, accden.at[didx], add=True)
        return carry

    lax.fori_loop(0, NCH, chunk_body, 0)

    plsc.subcore_barrier()
    pltpu.sync_copy(accnum.at[pl.ds(rbase, ROWS_PER_TILE)],
                    num_out.at[cid, pl.ds(rbase, ROWS_PER_TILE)])
    pltpu.sync_copy(accden.at[pl.ds(rbase, ROWS_PER_TILE)],
                    den_out.at[cid, pl.ds(rbase, ROWS_PER_TILE)])


_sc_edge = functools.partial(
    pl.kernel,
    _sc_edge_body,
    out_type=(jax.ShapeDtypeStruct((NC, NP, HC), _f32),
              jax.ShapeDtypeStruct((NC, NP, L), _f32)),
    mesh=plsc.VectorSubcoreMesh(core_axis_name="c", subcore_axis_name="s",
                                num_cores=NC, num_subcores=NS),
    compiler_params=pltpu.CompilerParams(use_tc_tiling_on_sc=False),
    scratch_types=(
        [pltpu.VMEM((K,), jnp.int32)]
        + [pltpu.VMEM((K,), jnp.int32)]
        + [pltpu.VMEM((K,), _f32)]
        + [pltpu.VMEM((K, L), _f32)]
        + [pltpu.VMEM((K, L), _f32)]
        + [pltpu.VMEM((K, HC), _f32)]
        + [pltpu.VMEM((K, L), _f32)]
        + [pltpu.VMEM((L,), _f32)]
        + [pltpu.VMEM_SHARED((NP, HC), _f32)]
        + [pltpu.VMEM_SHARED((NP, L), _f32)]
        + [pltpu.SemaphoreType.DMA] * 3
    ),
)()


def _combine_body(num_ref, den_ref, sel_ref, b_ref, out_ref):
    dsum = (den_ref[0] + den_ref[1])[:, :HP]
    denb = jnp.dot(dsum, jnp.transpose(sel_ref[...]),
                   preferred_element_type=_f32)
    out_ref[...] = (num_ref[0] + num_ref[1]) / denb + b_ref[...]


def _combine(num, den, sel, b2d):
    nblk = 8
    blk = NP // nblk
    return pl.pallas_call(
        _combine_body,
        grid=(nblk,),
        in_specs=[
            pl.BlockSpec((NC, blk, HC), lambda i: (0, i, 0)),
            pl.BlockSpec((NC, blk, L), lambda i: (0, i, 0)),
            pl.BlockSpec((HC, HP), lambda i: (0, 0)),
            pl.BlockSpec((1, HC), lambda i: (0, 0)),
        ],
        out_specs=pl.BlockSpec((blk, HC), lambda i: (i, 0)),
        out_shape=jax.ShapeDtypeStruct((NP, HC), _f32),
    )(num, den, sel, b2d)


def _layer(xp, src_p, dst_p, ew_p, w, att_src, att_dst, w_edge, att_edge,
           bias, sel, meanw):
    msrc = sel * att_src.reshape(-1)[:, None]
    mdst = sel * att_dst.reshape(-1)[:, None]
    wprod = (w_edge.reshape(1, HC) * att_edge.reshape(1, HC))
    h, asrc, adst, wc, snh, sdh = _prep(xp, w, msrc, mdst, wprod, sel, meanw)
    num, den = _sc_edge(src_p, dst_p, ew_p, asrc, adst, h,
                        jnp.tile(wc.reshape(HP), 2), snh, sdh)
    return _combine(num, den, sel, bias.reshape(1, HC))


def kernel(x, edge_index, edge_weight, W1, att_src1, att_dst1, W_edge1,
           att_edge1, bias1, W2, att_src2, att_dst2, W_edge2, att_edge2,
           bias2):
    src, dst = edge_index[0], edge_index[1]
    pad = EP - E
    src_p = jnp.concatenate([src, jnp.zeros((pad,), jnp.int32)])
    dst_p = jnp.concatenate([dst, jnp.full((pad,), PAD_DST, jnp.int32)])
    ew_p = jnp.concatenate([edge_weight, jnp.zeros((pad,), _f32)])
    xp = jnp.pad(x, ((0, NP - N), (0, 0)))
    sel = (jnp.arange(HC)[:, None] // C == jnp.arange(HP)[None, :]
           ).astype(_f32)
    meanw = _mean_ew(edge_weight)
    out1 = _layer(xp, src_p, dst_p, ew_p, W1, att_src1, att_dst1, W_edge1,
                  att_edge1, bias1, sel, meanw)
    out2 = _layer(out1, src_p, dst_p, ew_p, W2, att_src2, att_dst2, W_edge2,
                  att_edge2, bias2, sel, meanw)
    return out2[:N]


# speedup vs baseline: 1.6345x; 1.2565x over previous
import functools

import jax
import jax.numpy as jnp
from jax import lax
from jax.experimental import pallas as pl
from jax.experimental.pallas import tpu as pltpu
from jax.experimental.pallas import tpu_sc as plsc

N = 10000
NP = 10240
E = 640000
IN = 128
H = 4
C = 32
HC = H * C
HP = 8
NC, NS, L = 2, 16, 16
NW = NC * NS
K = 128
NCH = 157
EW = NCH * K
EP = EW * NW
ROWS_PER_TILE = NP // NS
PAD_DST = N + 100

_f32 = jnp.float32


def _ewsum_body(ew_ref, out_ref):
    @pl.when(pl.program_id(0) == 0)
    def _():
        out_ref[...] = jnp.zeros_like(out_ref)

    out_ref[...] = out_ref[...] + jnp.sum(ew_ref[...]).reshape(1, 1)


def _mean_ew(ew):
    ew2 = ew.reshape(5000, 128)
    s = pl.pallas_call(
        _ewsum_body,
        grid=(5,),
        in_specs=[pl.BlockSpec((1000, 128), lambda i: (i, 0))],
        out_specs=pl.BlockSpec((1, 1), lambda i: (0, 0)),
        out_shape=jax.ShapeDtypeStruct((1, 1), _f32),
    )(ew2)
    return s / float(E)


def _prep_body(x_ref, w_ref, msrc_ref, mdst_ref, wprod_ref, sel_ref,
               meanw_ref, h_ref, asrc_ref, adst_ref, wc_ref, snh_ref,
               sdh_ref):
    h = jnp.dot(x_ref[...], w_ref[...], preferred_element_type=_f32)
    h_ref[...] = h
    asrc = jnp.dot(h, msrc_ref[...], preferred_element_type=_f32)
    adst = jnp.dot(h, mdst_ref[...], preferred_element_type=_f32)
    asrc_ref[...] = jnp.concatenate([asrc, asrc], axis=1)
    adst_ref[...] = jnp.concatenate([adst, adst], axis=1)
    wc = jnp.dot(wprod_ref[...], sel_ref[...], preferred_element_type=_f32)
    wc_ref[...] = wc
    al = asrc + adst + meanw_ref[0, 0] * wc
    al = jnp.maximum(al, 0.2 * al)
    tl = jnp.exp(al)
    tlb = jnp.dot(tl, jnp.transpose(sel_ref[...]),
                  preferred_element_type=_f32)
    snh_ref[...] = 0.5 * h * tlb
    sdh_ref[...] = 0.5 * jnp.concatenate([tl, tl], axis=1)


def _prep(xp, w, msrc, mdst, wprod, sel, meanw):
    nblk = 8
    blk = NP // nblk
    return pl.pallas_call(
        _prep_body,
        grid=(nblk,),
        in_specs=[
            pl.BlockSpec((blk, IN), lambda i: (i, 0)),
            pl.BlockSpec((IN, HC), lambda i: (0, 0)),
            pl.BlockSpec((HC, HP), lambda i: (0, 0)),
            pl.BlockSpec((HC, HP), lambda i: (0, 0)),
            pl.BlockSpec((1, HC), lambda i: (0, 0)),
            pl.BlockSpec((HC, HP), lambda i: (0, 0)),
            pl.BlockSpec((1, 1), lambda i: (0, 0)),
        ],
        out_specs=[
            pl.BlockSpec((blk, HC), lambda i: (i, 0)),
            pl.BlockSpec((blk, L), lambda i: (i, 0)),
            pl.BlockSpec((blk, L), lambda i: (i, 0)),
            pl.BlockSpec((1, HP), lambda i: (0, 0)),
            pl.BlockSpec((blk, HC), lambda i: (i, 0)),
            pl.BlockSpec((blk, L), lambda i: (i, 0)),
        ],
        out_shape=[
            jax.ShapeDtypeStruct((NP, HC), _f32),
            jax.ShapeDtypeStruct((NP, L), _f32),
            jax.ShapeDtypeStruct((NP, L), _f32),
            jax.ShapeDtypeStruct((1, HP), _f32),
            jax.ShapeDtypeStruct((NP, HC), _f32),
            jax.ShapeDtypeStruct((NP, L), _f32),
        ],
    )(xp, w, msrc, mdst, wprod, sel, meanw)


def _sc_edge_body(src_hbm, dst_hbm, ew_hbm, asrc_hbm, adst_hbm, h_hbm,
                  wc_hbm, snh_hbm, sdh_hbm, num_out, den_out,
                  sidx, didx, ewb, asg, adg, hg, tb, wcv, accnum, accden,
                  sem0, sem1, sem2):
    cid = lax.axis_index("c")
    sid = lax.axis_index("s")
    wid = sid * NC + cid
    rbase = sid * ROWS_PER_TILE

    pltpu.sync_copy(snh_hbm.at[pl.ds(rbase, ROWS_PER_TILE)],
                    accnum.at[pl.ds(rbase, ROWS_PER_TILE)])
    pltpu.sync_copy(sdh_hbm.at[pl.ds(rbase, ROWS_PER_TILE)],
                    accden.at[pl.ds(rbase, ROWS_PER_TILE)])
    pltpu.sync_copy(wc_hbm, wcv)
    plsc.subcore_barrier()

    wc16 = wcv[...]
    ebase = wid * EW

    def chunk_body(i, carry):
        off = ebase + i * K
        d0 = pltpu.async_copy(src_hbm.at[pl.ds(off, K)], sidx, sem0)
        d1 = pltpu.async_copy(dst_hbm.at[pl.ds(off, K)], didx, sem1)
        d2 = pltpu.async_copy(ew_hbm.at[pl.ds(off, K)], ewb, sem2)
        d0.wait()
        d1.wait()
        d2.wait()
        g0 = pltpu.async_copy(asrc_hbm.at[sidx], asg, sem0)
        g1 = pltpu.async_copy(adst_hbm.at[didx], adg, sem1)
        g2 = pltpu.async_copy(h_hbm.at[sidx], hg, sem2)
        g0.wait()
        g1.wait()
        g2.wait()

        def t_body(g, _):
            ewv = ewb[pl.ds(g * L, L)]
            for j in range(L):
                e = g * L + j
                a = asg[e, :] + adg[e, :] + ewv[j] * wc16
                a = jnp.maximum(a, 0.2 * a)
                tb[e, :] = jnp.exp(a)
            return 0

        lax.fori_loop(0, K // L, t_body, 0)

        def s_body(e, _):
            tv = tb[e, :]
            for half in range(4):
                ts = tv[half]
                hg[e, pl.ds(half * 32, 16)] = \
                    hg[e, pl.ds(half * 32, 16)] * ts
                hg[e, pl.ds(half * 32 + 16, 16)] = \
                    hg[e, pl.ds(half * 32 + 16, 16)] * ts
            return 0

        lax.fori_loop(0, K, s_body, 0, unroll=2)

        pltpu.sync_copy(hg, accnum.at[didx], add=True)
        pltpu.sync_copy(tb, accden.at[didx], add=True)
        return carry

    lax.fori_loop(0, NCH, chunk_body, 0)

    plsc.subcore_barrier()
    pltpu.sync_copy(accnum.at[pl.ds(rbase, ROWS_PER_TILE)],
                    num_out.at[cid, pl.ds(rbase, ROWS_PER_TILE)])
    pltpu.sync_copy(accden.at[pl.ds(rbase, ROWS_PER_TILE)],
                    den_out.at[cid, pl.ds(rbase, ROWS_PER_TILE)])


_sc_edge = functools.partial(
    pl.kernel,
    _sc_edge_body,
    out_type=(jax.ShapeDtypeStruct((NC, NP, HC), _f32),
              jax.ShapeDtypeStruct((NC, NP, L), _f32)),
    mesh=plsc.VectorSubcoreMesh(core_axis_name="c", subcore_axis_name="s",
                                num_cores=NC, num_subcores=NS),
    compiler_params=pltpu.CompilerParams(use_tc_tiling_on_sc=False),
    scratch_types=(
        [pltpu.VMEM((K,), jnp.int32)]
        + [pltpu.VMEM((K,), jnp.int32)]
        + [pltpu.VMEM((K,), _f32)]
        + [pltpu.VMEM((K, L), _f32)]
        + [pltpu.VMEM((K, L), _f32)]
        + [pltpu.VMEM((K, HC), _f32)]
        + [pltpu.VMEM((K, L), _f32)]
        + [pltpu.VMEM((L,), _f32)]
        + [pltpu.VMEM_SHARED((NP, HC), _f32)]
        + [pltpu.VMEM_SHARED((NP, L), _f32)]
        + [pltpu.SemaphoreType.DMA] * 3
    ),
)()


def _combine_body(num_ref, den_ref, sel_ref, b_ref, out_ref):
    dsum = (den_ref[0] + den_ref[1])[:, :HP]
    denb = jnp.dot(dsum, jnp.transpose(sel_ref[...]),
                   preferred_element_type=_f32)
    out_ref[...] = (num_ref[0] + num_ref[1]) / denb + b_ref[...]


def _combine(num, den, sel, b2d):
    nblk = 8
    blk = NP // nblk
    return pl.pallas_call(
        _combine_body,
        grid=(nblk,),
        in_specs=[
            pl.BlockSpec((NC, blk, HC), lambda i: (0, i, 0)),
            pl.BlockSpec((NC, blk, L), lambda i: (0, i, 0)),
            pl.BlockSpec((HC, HP), lambda i: (0, 0)),
            pl.BlockSpec((1, HC), lambda i: (0, 0)),
        ],
        out_specs=pl.BlockSpec((blk, HC), lambda i: (i, 0)),
        out_shape=jax.ShapeDtypeStruct((NP, HC), _f32),
    )(num, den, sel, b2d)


def _layer(xp, src_p, dst_p, ew_p, w, att_src, att_dst, w_edge, att_edge,
           bias, sel, meanw):
    msrc = sel * att_src.reshape(-1)[:, None]
    mdst = sel * att_dst.reshape(-1)[:, None]
    wprod = (w_edge.reshape(1, HC) * att_edge.reshape(1, HC))
    h, asrc, adst, wc, snh, sdh = _prep(xp, w, msrc, mdst, wprod, sel, meanw)
    num, den = _sc_edge(src_p, dst_p, ew_p, asrc, adst, h,
                        jnp.tile(wc.reshape(HP), 2), snh, sdh)
    return _combine(num, den, sel, bias.reshape(1, HC))


def kernel(x, edge_index, edge_weight, W1, att_src1, att_dst1, W_edge1,
           att_edge1, bias1, W2, att_src2, att_dst2, W_edge2, att_edge2,
           bias2):
    src, dst = edge_index[0], edge_index[1]
    pad = EP - E
    src_p = jnp.concatenate([src, jnp.zeros((pad,), jnp.int32)])
    dst_p = jnp.concatenate([dst, jnp.full((pad,), PAD_DST, jnp.int32)])
    ew_p = jnp.concatenate([edge_weight, jnp.zeros((pad,), _f32)])
    xp = jnp.pad(x, ((0, NP - N), (0, 0)))
    sel = (jnp.arange(HC)[:, None] // C == jnp.arange(HP)[None, :]
           ).astype(_f32)
    meanw = _mean_ew(edge_weight)
    out1 = _layer(xp, src_p, dst_p, ew_p, W1, att_src1, att_dst1, W_edge1,
                  att_edge1, bias1, sel, meanw)
    out2 = _layer(out1, src_p, dst_p, ew_p, W2, att_src2, att_dst2, W_edge2,
                  att_edge2, bias2, sel, meanw)
    return out2[:N]
